# baseline (speedup 1.0000x reference)
import functools

import jax
import jax.numpy as jnp
from jax import lax
from jax.experimental import pallas as pl
from jax.experimental.pallas import tpu as pltpu
from jax.experimental.pallas import tpu_sc as plsc

NUM_CONN = 32
LANES = 128

SC_COLS = 2048
SC_NC = 2
SC_NS = 16
SC_CC = 32
SC_OB = 256


def _tree_reduce(vals, combine):
    while len(vals) > 1:
        nxt = [combine(vals[j], vals[j + 1]) for j in range(0, len(vals) - 1, 2)]
        if len(vals) % 2:
            nxt.append(vals[-1])
        vals = nxt
    return vals[0]


_OPS = (
    (jnp.minimum, lambda v: v, lambda v: v),
    (jnp.maximum, lambda v: v, lambda v: v),
    (lax.mul, lambda v: v, lambda v: v),
    (lax.mul, lambda v: 1.0 - v, lambda v: 1.0 - v),
)


def _ddlg_body(conn_ref, order_ref, starts_ref, x_hbm, o_ref, x_ref, sem,
               *, half, g):
    i = pl.program_id(0)
    j = pl.program_id(1)

    @pl.when(j == 0)
    def _():
        cp = pltpu.make_async_copy(
            x_hbm.at[:, pl.ds(i * g, g), :], x_ref, sem)
        cp.start()
        cp.wait()

    def segment(opcode, combine, leaf, fin):
        lo = starts_ref[j, opcode]
        hi = starts_ref[j, opcode + 1]

        def it(ii, carry):
            base = ii * NUM_CONN
            rows = [leaf(x_ref[conn_ref[base + k]]) for k in range(NUM_CONN)]
            o_ref[order_ref[ii] - j * half] = fin(_tree_reduce(rows, combine))
            return carry

        lax.fori_loop(lo, hi, it, 0)

    for opcode, (combine, leaf, fin) in enumerate(_OPS):
        segment(opcode, combine, leaf, fin)


@functools.partial(jax.jit, static_argnames=("out_f", "g", "out_split"))
def _ddlg_layer(x3, conn_s, order, starts, out_f, g, out_split):
    in_f, nb, _ = x3.shape
    half = out_f // out_split
    grid = (nb // g, out_split)
    return pl.pallas_call(
        functools.partial(_ddlg_body, half=half, g=g),
        grid_spec=pltpu.PrefetchScalarGridSpec(
            num_scalar_prefetch=3,
            grid=grid,
            in_specs=[pl.BlockSpec(memory_space=pltpu.MemorySpace.HBM)],
            out_specs=pl.BlockSpec((half, g, LANES), lambda i, j, *_: (j, i, 0)),
            scratch_shapes=[
                pltpu.VMEM((in_f, g, LANES), jnp.float32),
                pltpu.SemaphoreType.DMA,
            ],
        ),
        out_shape=jax.ShapeDtypeStruct((out_f, nb, LANES), jnp.float32),
        compiler_params=pltpu.CompilerParams(
            dimension_semantics=("arbitrary", "arbitrary"),
        ),
    )(conn_s, order, starts, x3)


def _group_by_block_opcode(W, conn, nblocks, block):
    opcode = jnp.argmax(W, axis=-1).astype(jnp.int32)
    out_f = W.shape[0]
    rows = jnp.arange(out_f, dtype=jnp.int32)
    key = (rows // block) * 4 + opcode
    order = jnp.argsort(key, stable=True).astype(jnp.int32)
    bounds = jnp.searchsorted(
        key[order], jnp.arange(nblocks * 4 + 1, dtype=jnp.int32), side="left"
    ).astype(jnp.int32)
    starts = bounds[jnp.arange(nblocks)[:, None] * 4 + jnp.arange(5)[None, :]]
    conn_s = conn[order].reshape(-1)
    return conn_s, order, starts


def _tc_chain(x_rows, Ws, conns):
    batch, in_f = x_rows.shape
    nb = batch // LANES
    h = jnp.transpose(x_rows).reshape(in_f, nb, LANES)
    for W, conn in zip(Ws, conns):
        out_f = W.shape[0]
        if nb % 48 == 0:
            g, out_split = 48, 32
        elif nb % 32 == 0:
            g, out_split = 32, 8
        else:
            g, out_split = min(16, nb), 8
        conn_s, order, starts = _group_by_block_opcode(
            W, conn, out_split, out_f // out_split)
        h = _ddlg_layer(h, conn_s, order, starts, out_f=out_f, g=g,
                        out_split=out_split)
    out_f = h.shape[0]
    return jnp.transpose(h.reshape(out_f, batch))


SC_REC = 40
SC_BLK = SC_OB * SC_REC + 16


@functools.partial(jax.jit, static_argnames=("out_f", "cc"))
def _sc_layer(x2, recs, bounds, out_f, cc):
    nch, in_f_cc = x2.shape
    in_f = in_f_cc // cc
    nw = SC_NC * SC_NS
    cpw = nch // nw
    nblocks = out_f // SC_OB
    mesh = plsc.VectorSubcoreMesh(core_axis_name="c", subcore_axis_name="s")

    @functools.partial(
        pl.kernel,
        mesh=mesh,
        out_type=jax.ShapeDtypeStruct((nch, out_f, cc), jnp.float32),
        scratch_types=[
            pltpu.VMEM((in_f * cc,), jnp.float32),
            pltpu.VMEM((SC_BLK,), jnp.int32),
            pltpu.VMEM((nblocks * 8 + 16,), jnp.int32),
            pltpu.VMEM((SC_OB, cc), jnp.float32),
        ],
    )
    def k(x_hbm, recs_hbm, bounds_hbm, out_hbm, xv, recv, bndv, bufv):
        cid = lax.axis_index("c")
        sid = lax.axis_index("s")
        wid = sid * SC_NC + cid
        pltpu.sync_copy(bounds_hbm, bndv)

        def chunk_body(t, carry):
            ch = wid * cpw + t
            pltpu.sync_copy(x_hbm.at[ch], xv)

            def block_body(ob, carry2):
                pltpu.sync_copy(recs_hbm.at[pl.ds(ob * SC_BLK, SC_BLK)], recv)
                bv = bndv[pl.ds(ob * 8, 16)]

                def segment(seg, combine, leaf, fin):
                    lo = bv[seg]
                    hi = bv[seg + 1]

                    @plsc.parallel_loop(lo, hi, unroll=4)
                    def _(i):
                        base = i * SC_REC
                        iv0 = recv[pl.ds(base, 16)]
                        iv1 = recv[pl.ds(base + 16, 16)]
                        iv2 = recv[pl.ds(base + 32, 16)]
                        idxs = ([iv0[kk] for kk in range(16)]
                                + [iv1[kk] for kk in range(16)])
                        d = iv2[0]
                        for h in range(cc // 16):
                            sl = h * 16
                            rows = [leaf(xv[pl.ds(idxs[kk] + sl, 16)])
                                    for kk in range(NUM_CONN)]
                            bufv[d, pl.ds(sl, 16)] = fin(
                                _tree_reduce(rows, combine))

                for seg, (combine, leaf, fin) in enumerate(_OPS):
                    segment(seg, combine, leaf, fin)
                pltpu.sync_copy(bufv, out_hbm.at[ch, pl.ds(ob * SC_OB, SC_OB)])
                return carry2

            lax.fori_loop(0, nblocks, block_body, 0)
            return carry

        lax.fori_loop(0, cpw, chunk_body, 0)

    return k(x2, recs, bounds)


def _sc_chain(x_rows, Ws, conns):
    bsc, in_f = x_rows.shape
    nch = bsc // SC_CC
    h = jnp.transpose(x_rows).reshape(in_f, nch, SC_CC).transpose(1, 0, 2)
    for W, conn in zip(Ws, conns):
        out_f = W.shape[0]
        nblocks = out_f // SC_OB
        conn_s, order, starts = _group_by_block_opcode(W, conn, nblocks, SC_OB)
        ldest_s = order - (jnp.arange(out_f, dtype=jnp.int32) // SC_OB) * SC_OB
        starts_loc = (starts
                      - (jnp.arange(nblocks, dtype=jnp.int32) * SC_OB)[:, None])
        recs = jnp.concatenate(
            [conn_s.reshape(out_f, NUM_CONN) * SC_CC, ldest_s[:, None],
             jnp.zeros((out_f, SC_REC - NUM_CONN - 1), jnp.int32)], axis=1)
        recs = recs.reshape(nblocks, SC_OB * SC_REC)
        recs = jnp.concatenate(
            [recs, jnp.zeros((nblocks, 16), jnp.int32)], axis=1).reshape(-1)
        bounds = jnp.concatenate(
            [starts_loc, jnp.zeros((nblocks, 3), jnp.int32)], axis=1)
        bounds = jnp.concatenate(
            [bounds.reshape(-1), jnp.zeros((16,), jnp.int32)])
        h = _sc_layer(h.reshape(nch, -1), recs, bounds, out_f=out_f, cc=SC_CC)
    out_f = h.shape[1]
    return jnp.transpose(h, (1, 0, 2)).reshape(out_f, bsc).T


def kernel(x, W0, W1, W2, W3, conn0, conn1, conn2, conn3):
    Ws = [W0, W1, W2, W3]
    conns = [conn0, conn1, conn2, conn3]
    batch, _ = x.shape
    sc_cols = min(SC_COLS, batch)
    parts = []
    if batch - sc_cols:
        parts.append(_tc_chain(x[:batch - sc_cols], Ws, conns))
    if sc_cols:
        parts.append(_sc_chain(x[batch - sc_cols:], Ws, conns))
    return parts[0] if len(parts) == 1 else jnp.concatenate(parts, axis=0)


# speedup vs baseline: 1.0871x; 1.0871x over previous
import functools

import jax
import jax.numpy as jnp
from jax import lax
from jax.experimental import pallas as pl
from jax.experimental.pallas import tpu as pltpu
from jax.experimental.pallas import tpu_sc as plsc

NUM_CONN = 32
LANES = 128

SC_COLS = 2048
SC_NC = 2
SC_NS = 16
SC_CC = 32
SC_OB = 256


def _tree_reduce(vals, combine):
    while len(vals) > 1:
        nxt = [combine(vals[j], vals[j + 1]) for j in range(0, len(vals) - 1, 2)]
        if len(vals) % 2:
            nxt.append(vals[-1])
        vals = nxt
    return vals[0]


_OPS = (
    (jnp.minimum, lambda v: v, lambda v: v),
    (jnp.maximum, lambda v: v, lambda v: v),
    (lax.mul, lambda v: v, lambda v: v),
    (lax.mul, lambda v: 1.0 - v, lambda v: 1.0 - v),
)


def _ddlg_body(conn_ref, order_ref, starts_ref, x_hbm, o_ref, x_ref, sem,
               *, half, g):
    i = pl.program_id(0)
    j = pl.program_id(1)

    @pl.when(j == 0)
    def _():
        cp = pltpu.make_async_copy(
            x_hbm.at[:, pl.ds(i * g, g), :], x_ref, sem)
        cp.start()
        cp.wait()

    def segment(opcode, combine, leaf, fin):
        lo = starts_ref[j, opcode]
        hi = starts_ref[j, opcode + 1]

        def it(ii, carry):
            base = ii * NUM_CONN
            rows = [leaf(x_ref[conn_ref[base + k]]) for k in range(NUM_CONN)]
            o_ref[order_ref[ii] - j * half] = fin(_tree_reduce(rows, combine))
            return carry

        lax.fori_loop(lo, hi, it, 0)

    for opcode, (combine, leaf, fin) in enumerate(_OPS):
        segment(opcode, combine, leaf, fin)


@functools.partial(jax.jit, static_argnames=("out_f", "g", "out_split"))
def _ddlg_layer(x3, conn_s, order, starts, out_f, g, out_split):
    in_f, nb, _ = x3.shape
    half = out_f // out_split
    grid = (nb // g, out_split)
    return pl.pallas_call(
        functools.partial(_ddlg_body, half=half, g=g),
        grid_spec=pltpu.PrefetchScalarGridSpec(
            num_scalar_prefetch=3,
            grid=grid,
            in_specs=[pl.BlockSpec(memory_space=pltpu.MemorySpace.HBM)],
            out_specs=pl.BlockSpec((half, g, LANES), lambda i, j, *_: (j, i, 0)),
            scratch_shapes=[
                pltpu.VMEM((in_f, g, LANES), jnp.float32),
                pltpu.SemaphoreType.DMA,
            ],
        ),
        out_shape=jax.ShapeDtypeStruct((out_f, nb, LANES), jnp.float32),
        compiler_params=pltpu.CompilerParams(
            dimension_semantics=("arbitrary", "arbitrary"),
        ),
    )(conn_s, order, starts, x3)


def _group_by_block_opcode(W, conn, nblocks, block):
    opcode = jnp.argmax(W, axis=-1).astype(jnp.int32)
    out_f = W.shape[0]
    rows = jnp.arange(out_f, dtype=jnp.int32)
    key = (rows // block) * 4 + opcode
    order = jnp.argsort(key, stable=True).astype(jnp.int32)
    bounds = jnp.searchsorted(
        key[order], jnp.arange(nblocks * 4 + 1, dtype=jnp.int32), side="left"
    ).astype(jnp.int32)
    starts = bounds[jnp.arange(nblocks)[:, None] * 4 + jnp.arange(5)[None, :]]
    conn_s = conn[order].reshape(-1)
    return conn_s, order, starts


def _tc_chain(x_rows, Ws, conns):
    batch, in_f = x_rows.shape
    nb = batch // LANES
    h = jnp.transpose(x_rows).reshape(in_f, nb, LANES)
    for W, conn in zip(Ws, conns):
        out_f = W.shape[0]
        if nb % 48 == 0:
            g, out_split = 48, 32
        elif nb % 32 == 0:
            g, out_split = 32, 8
        else:
            g, out_split = min(16, nb), 8
        conn_s, order, starts = _group_by_block_opcode(
            W, conn, out_split, out_f // out_split)
        h = _ddlg_layer(h, conn_s, order, starts, out_f=out_f, g=g,
                        out_split=out_split)
    out_f = h.shape[0]
    return jnp.transpose(h.reshape(out_f, batch))


SC_REC = 40
SC_BLK = SC_OB * SC_REC + 16


@functools.partial(jax.jit, static_argnames=("out_f", "cc"))
def _sc_layer(x2, recs, bounds, out_f, cc):
    nch, in_f_cc = x2.shape
    in_f = in_f_cc // cc
    nw = SC_NC * SC_NS
    cpw = nch // nw
    nblocks = out_f // SC_OB
    mesh = plsc.VectorSubcoreMesh(core_axis_name="c", subcore_axis_name="s")

    @functools.partial(
        pl.kernel,
        mesh=mesh,
        out_type=jax.ShapeDtypeStruct((nch, out_f, cc), jnp.float32),
        scratch_types=[
            pltpu.VMEM((in_f * cc,), jnp.float32),
            pltpu.VMEM((SC_BLK,), jnp.int32),
            pltpu.VMEM((nblocks * 8 + 16,), jnp.int32),
            pltpu.VMEM((SC_OB, cc), jnp.float32),
        ],
    )
    def k(x_hbm, recs_hbm, bounds_hbm, out_hbm, xv, recv, bndv, bufv):
        cid = lax.axis_index("c")
        sid = lax.axis_index("s")
        wid = sid * SC_NC + cid
        pltpu.sync_copy(bounds_hbm, bndv)

        def chunk_body(t, carry):
            ch = wid * cpw + t
            pltpu.sync_copy(x_hbm.at[ch], xv)

            def block_body(ob, carry2):
                pltpu.sync_copy(recs_hbm.at[pl.ds(ob * SC_BLK, SC_BLK)], recv)
                bv = bndv[pl.ds(ob * 8, 16)]

                def segment(seg, combine, leaf, fin):
                    lo = bv[seg]
                    hi = bv[seg + 1]

                    @plsc.parallel_loop(lo, hi, unroll=2)
                    def _(i):
                        base = i * SC_REC
                        iv0 = recv[pl.ds(base, 16)]
                        iv1 = recv[pl.ds(base + 16, 16)]
                        iv2 = recv[pl.ds(base + 32, 16)]
                        idxs = ([iv0[kk] for kk in range(16)]
                                + [iv1[kk] for kk in range(16)])
                        d = iv2[0]
                        for h in range(cc // 16):
                            sl = h * 16
                            rows = [leaf(xv[pl.ds(idxs[kk] + sl, 16)])
                                    for kk in range(NUM_CONN)]
                            bufv[d, pl.ds(sl, 16)] = fin(
                                _tree_reduce(rows, combine))

                for seg, (combine, leaf, fin) in enumerate(_OPS):
                    segment(seg, combine, leaf, fin)
                pltpu.sync_copy(bufv, out_hbm.at[ch, pl.ds(ob * SC_OB, SC_OB)])
                return carry2

            lax.fori_loop(0, nblocks, block_body, 0)
            return carry

        lax.fori_loop(0, cpw, chunk_body, 0)

    return k(x2, recs, bounds)


def _sc_chain(x_rows, Ws, conns):
    bsc, in_f = x_rows.shape
    nch = bsc // SC_CC
    h = jnp.transpose(x_rows).reshape(in_f, nch, SC_CC).transpose(1, 0, 2)
    for W, conn in zip(Ws, conns):
        out_f = W.shape[0]
        nblocks = out_f // SC_OB
        conn_s, order, starts = _group_by_block_opcode(W, conn, nblocks, SC_OB)
        ldest_s = order - (jnp.arange(out_f, dtype=jnp.int32) // SC_OB) * SC_OB
        starts_loc = (starts
                      - (jnp.arange(nblocks, dtype=jnp.int32) * SC_OB)[:, None])
        recs = jnp.concatenate(
            [conn_s.reshape(out_f, NUM_CONN) * SC_CC, ldest_s[:, None],
             jnp.zeros((out_f, SC_REC - NUM_CONN - 1), jnp.int32)], axis=1)
        recs = recs.reshape(nblocks, SC_OB * SC_REC)
        recs = jnp.concatenate(
            [recs, jnp.zeros((nblocks, 16), jnp.int32)], axis=1).reshape(-1)
        bounds = jnp.concatenate(
            [starts_loc, jnp.zeros((nblocks, 3), jnp.int32)], axis=1)
        bounds = jnp.concatenate(
            [bounds.reshape(-1), jnp.zeros((16,), jnp.int32)])
        h = _sc_layer(h.reshape(nch, -1), recs, bounds, out_f=out_f, cc=SC_CC)
    out_f = h.shape[1]
    return jnp.transpose(h, (1, 0, 2)).reshape(out_f, bsc).T


def kernel(x, W0, W1, W2, W3, conn0, conn1, conn2, conn3):
    Ws = [W0, W1, W2, W3]
    conns = [conn0, conn1, conn2, conn3]
    batch, _ = x.shape
    sc_cols = min(SC_COLS, batch)
    parts = []
    if batch - sc_cols:
        parts.append(_tc_chain(x[:batch - sc_cols], Ws, conns))
    if sc_cols:
        parts.append(_sc_chain(x[batch - sc_cols:], Ws, conns))
    return parts[0] if len(parts) == 1 else jnp.concatenate(parts, axis=0)
